# baseline (speedup 1.0000x reference)
import jax
import jax.numpy as jnp
from jax.experimental import pallas as pl
from jax.experimental.pallas import tpu as pltpu

K = 16
QB = 128
CB = 1024
BIG = 2**30


def _extract_topk(d, io, nk):
    kiota = jax.lax.broadcasted_iota(jnp.int32, (nk, d.shape[1]), 0)
    cv = jnp.zeros((nk, d.shape[1]), jnp.float32)
    cg = jnp.zeros((nk, d.shape[1]), jnp.int32)
    for k in range(nk):
        m = jnp.min(d, axis=0, keepdims=True)
        gi = jnp.min(jnp.where(d == m, io, BIG), axis=0, keepdims=True)
        cv = jnp.where(kiota == k, m, cv)
        cg = jnp.where(kiota == k, gi, cg)
        d = jnp.where(io == gi, jnp.float32(jnp.inf), d)
    return cv, cg


def _knn_kernel(q_ref, s_ref, out_ref, bv_ref, bg_ref):
    c = pl.program_id(2)

    @pl.when(c == 0)
    def init():
        bv_ref[...] = jnp.full(bv_ref.shape, jnp.inf, jnp.float32)
        bg_ref[...] = jnp.full(bg_ref.shape, BIG, jnp.int32)

    q = q_ref[0]
    s = s_ref[0]
    sn = jnp.sum(s * s, axis=1, keepdims=True)
    qn = jnp.sum(q * q, axis=1)
    sq = jax.lax.dot_general(
        s, q, (((1,), (1,)), ((), ())),
        precision=jax.lax.Precision.HIGHEST,
        preferred_element_type=jnp.float32,
    )
    d = (sn - 2.0 * sq) + qn[None, :]
    io = jax.lax.broadcasted_iota(jnp.int32, d.shape, 0) + c * CB
    cv, cg = _extract_topk(d, io, K)

    vcomb = jnp.concatenate([bv_ref[...], cv], axis=0)
    gcomb = jnp.concatenate([bg_ref[...], cg], axis=0)
    bv, bg = _extract_topk(vcomb, gcomb, K)
    bv_ref[...] = bv
    bg_ref[...] = bg
    out_ref[0] = bg


def kernel(query_points, sample_points):
    B, Q, D = query_points.shape
    _, N, _ = sample_points.shape
    grid = (B, Q // QB, N // CB)
    out = pl.pallas_call(
        _knn_kernel,
        grid=grid,
        in_specs=[
            pl.BlockSpec((1, QB, D), lambda b, i, c: (b, i, 0)),
            pl.BlockSpec((1, CB, D), lambda b, i, c: (b, c, 0)),
        ],
        out_specs=pl.BlockSpec((1, K, QB), lambda b, i, c: (b, 0, i)),
        out_shape=jax.ShapeDtypeStruct((B, K, Q), jnp.int32),
        scratch_shapes=[
            pltpu.VMEM((K, QB), jnp.float32),
            pltpu.VMEM((K, QB), jnp.int32),
        ],
    )(query_points, sample_points)
    return jnp.transpose(out, (0, 2, 1))


# speedup vs baseline: 1.3631x; 1.3631x over previous
import jax
import jax.numpy as jnp
from jax.experimental import pallas as pl
from jax.experimental.pallas import tpu as pltpu

K = 16
QB = 128
CB = 512
BIG = 2**30


def _lexmin_reduce(v, i):
    r = v.shape[0]
    while r > 1:
        h = r // 2
        cmp = v[:h] <= v[h:]
        v = jnp.where(cmp, v[:h], v[h:])
        i = jnp.where(cmp, i[:h], i[h:])
        r = h
    return v, i


def _extract_topk(d, io, nk):
    kiota = jax.lax.broadcasted_iota(jnp.int32, (nk, d.shape[1]), 0)
    cv = jnp.zeros((nk, d.shape[1]), jnp.float32)
    cg = jnp.zeros((nk, d.shape[1]), jnp.int32)
    for k in range(nk):
        m, gi = _lexmin_reduce(d, io)
        cv = jnp.where(kiota == k, m, cv)
        cg = jnp.where(kiota == k, gi, cg)
        d = jnp.where(io == gi, jnp.float32(jnp.inf), d)
    return cv, cg


def _knn_kernel(q_ref, s_ref, out_ref, bv_ref, bg_ref):
    c = pl.program_id(2)

    @pl.when(c == 0)
    def init():
        bv_ref[...] = jnp.full(bv_ref.shape, jnp.inf, jnp.float32)
        bg_ref[...] = jnp.full(bg_ref.shape, BIG, jnp.int32)

    q = q_ref[0]
    s = s_ref[0]
    sn = jnp.sum(s * s, axis=1, keepdims=True)
    qn = jnp.sum(q * q, axis=1)
    sq = jax.lax.dot_general(
        s, q, (((1,), (1,)), ((), ())),
        precision=jax.lax.Precision.HIGHEST,
        preferred_element_type=jnp.float32,
    )
    d = (sn - 2.0 * sq) + qn[None, :]
    io = jax.lax.broadcasted_iota(jnp.int32, d.shape, 0) + c * CB
    cv, cg = _extract_topk(d, io, K)

    vcomb = jnp.concatenate([bv_ref[...], cv], axis=0)
    gcomb = jnp.concatenate([bg_ref[...], cg], axis=0)
    bv, bg = _extract_topk(vcomb, gcomb, K)
    bv_ref[...] = bv
    bg_ref[...] = bg
    out_ref[0] = bg


def kernel(query_points, sample_points):
    B, Q, D = query_points.shape
    _, N, _ = sample_points.shape
    grid = (B, Q // QB, N // CB)
    out = pl.pallas_call(
        _knn_kernel,
        grid=grid,
        in_specs=[
            pl.BlockSpec((1, QB, D), lambda b, i, c: (b, i, 0)),
            pl.BlockSpec((1, CB, D), lambda b, i, c: (b, c, 0)),
        ],
        out_specs=pl.BlockSpec((1, K, QB), lambda b, i, c: (b, 0, i)),
        out_shape=jax.ShapeDtypeStruct((B, K, Q), jnp.int32),
        scratch_shapes=[
            pltpu.VMEM((K, QB), jnp.float32),
            pltpu.VMEM((K, QB), jnp.int32),
        ],
    )(query_points, sample_points)
    return jnp.transpose(out, (0, 2, 1))
